# Initial kernel scaffold; baseline (speedup 1.0000x reference)
import jax
import jax.numpy as jnp
from jax.experimental import pallas as pl


def kernel(src, dst, neg, time, nbr_nids, nbr_times, nbr_feats, static_node_feat, wt, bt, W1, b1, W2, b2, fc1_w, fc1_b, fc2_w, fc2_b, P):
    raise NotImplementedError("write your pallas kernel here")


# speedup vs baseline: 3.5504x; 3.5504x over previous
import functools

import jax
import jax.numpy as jnp
from jax import lax
from jax.experimental import pallas as pl
from jax.experimental.pallas import tpu as pltpu
from jax.experimental.pallas import tpu_sc as plsc

NN = 100000
B = 4096
K = 20
D = 128
E = 16
T = 100
L1 = 3
RP = 64
HID = 128
OUT = 128

NW = 32
CH = 64

NFN = 3 * B * K
NFO = 3 * B
NPN = L1 * NFN
NPO = L1 * NFO

CFN = NFN // (NW * CH)
CFO = NFO // (NW * CH)
CPN = NPN // (NW * CH)
CPO = NPO // (NW * CH)

R = 128
JB = B // R

_f32 = jnp.float32


@functools.lru_cache(maxsize=1)
def _make_sc_gather():
    return functools.partial(
        pl.kernel,
        out_type=[
            jax.ShapeDtypeStruct((NFN, D), _f32),
            jax.ShapeDtypeStruct((NFO, D), _f32),
            jax.ShapeDtypeStruct((NPN, RP), _f32),
            jax.ShapeDtypeStruct((NPO, RP), _f32),
        ],
        mesh=plsc.VectorSubcoreMesh(core_axis_name="c", subcore_axis_name="s"),
        compiler_params=pltpu.CompilerParams(use_tc_tiling_on_sc=False),
        scratch_types=[
            pltpu.VMEM((CFN, CH), jnp.int32),
            pltpu.VMEM((CFO, CH), jnp.int32),
            pltpu.VMEM((CPN, CH), jnp.int32),
            pltpu.VMEM((CPO, CH), jnp.int32),
            pltpu.VMEM((CH, D), _f32),
            pltpu.VMEM((CH, RP), _f32),
            pltpu.SemaphoreType.DMA,
        ],
    )(_sc_gather_body)


def _sc_gather_body(feat_hbm, pflat_hbm, ifn_hbm, ifo_hbm, ipn_hbm, ipo_hbm,
                    ofn, ofo, opn, opo, ifn_v, ifo_v, ipn_v, ipo_v,
                    fbuf, pbuf, sem):
    wid = lax.axis_index("s") * 2 + lax.axis_index("c")
    pltpu.sync_copy(ifn_hbm.at[wid], ifn_v)
    pltpu.sync_copy(ifo_hbm.at[wid], ifo_v)
    pltpu.sync_copy(ipn_hbm.at[wid], ipn_v)
    pltpu.sync_copy(ipo_hbm.at[wid], ipo_v)

    def run(table, idx_v, nch, out, buf):
        base = wid * (nch * CH)

        def body(jc, carry):
            pltpu.async_copy(table.at[idx_v.at[jc]], buf, sem).wait()
            pltpu.sync_copy(buf, out.at[pl.ds(base + jc * CH, CH)])
            return carry

        lax.fori_loop(0, nch, body, 0)

    run(feat_hbm, ifn_v, CFN, ofn, fbuf)
    run(feat_hbm, ifo_v, CFO, ofo, fbuf)
    run(pflat_hbm, ipn_v, CPN, opn, pbuf)
    run(pflat_hbm, ipo_v, CPO, opo, pbuf)


def _tc_body(nbr_emb_ref, nbr_feats_ref, nbr_times_ref, time_ref,
             pn_ref, pu_ref, pv_ref, pv2_ref, ne_ref,
             wt_ref, bt_ref, w1a_ref, w1b_ref, w1c_ref, w1de_ref, b1_ref,
             w2a_ref, w2b_ref, b2_ref, fc1a_ref, fc1b_ref, fc1bias_ref,
             fc2r_ref, fc2b_ref,
             pos_ref, neg_ref,
             zsp_ref, zsn_ref, zdp_ref):
    s = pl.program_id(1)

    ne3 = nbr_emb_ref[...].reshape(R, K, D)
    nf3 = nbr_feats_ref[...].reshape(R, K, E)
    dt = time_ref[...] - nbr_times_ref[...]
    te = jnp.cos(dt[..., None] * wt_ref[...].reshape(1, 1, T)
                 + bt_ref[...].reshape(1, 1, T))

    dn = (((2,), (0,)), ((), ()))
    acc = (lax.dot_general(ne3, w1a_ref[...], dn, preferred_element_type=_f32)
           + lax.dot_general(nf3, w1b_ref[...], dn, preferred_element_type=_f32)
           + lax.dot_general(te, w1c_ref[...], dn, preferred_element_type=_f32)
           + b1_ref[...].reshape(1, 1, HID))

    pn = pn_ref[...].reshape(L1, R, K, RP)
    pu = pu_ref[...]
    pv = pv_ref[...]
    w1de = w1de_ref[...]

    g1s = []
    accp = acc
    for l in range(L1):
        g1 = (pn[l] * pu[l][:, None, :]).sum(-1)
        g1s.append(g1)
        g2 = (pn[l] * pv[l][:, None, :]).sum(-1)
        accp = accp + g1[..., None] * w1de[l].reshape(1, 1, HID)
        accp = accp + g2[..., None] * w1de[L1 + l].reshape(1, 1, HID)

    h = jnp.mean(jax.nn.relu(accp), axis=1)
    nemb = ne_ref[...]
    w2a = w2a_ref[...]
    w2b = w2b_ref[...]
    b2 = b2_ref[...]
    z = (jnp.dot(h, w2a, preferred_element_type=_f32)
         + jnp.dot(nemb, w2b, preferred_element_type=_f32) + b2)

    @pl.when(s == 0)
    def _():
        zsp_ref[...] = z
        pv2 = pv2_ref[...]
        accn = acc
        for l in range(L1):
            g2n = (pn[l] * pv2[l][:, None, :]).sum(-1)
            accn = accn + g1s[l][..., None] * w1de[l].reshape(1, 1, HID)
            accn = accn + g2n[..., None] * w1de[L1 + l].reshape(1, 1, HID)
        hn = jnp.mean(jax.nn.relu(accn), axis=1)
        zsn_ref[...] = (jnp.dot(hn, w2a, preferred_element_type=_f32)
                        + jnp.dot(nemb, w2b, preferred_element_type=_f32) + b2)

    @pl.when(s == 1)
    def _():
        zdp_ref[...] = z

    @pl.when(s == 2)
    def _():
        fc1a = fc1a_ref[...]
        fc1b = fc1b_ref[...]
        fc1bias = fc1bias_ref[...]
        fc2r = fc2r_ref[...]
        fc2b = fc2b_ref[...]
        hp = jax.nn.relu(jnp.dot(zsp_ref[...], fc1a, preferred_element_type=_f32)
                         + jnp.dot(zdp_ref[...], fc1b, preferred_element_type=_f32)
                         + fc1bias)
        pos_ref[...] = jax.nn.sigmoid((hp * fc2r).sum(1, keepdims=True) + fc2b)
        hn = jax.nn.relu(jnp.dot(zsn_ref[...], fc1a, preferred_element_type=_f32)
                         + jnp.dot(z, fc1b, preferred_element_type=_f32)
                         + fc1bias)
        neg_ref[...] = jax.nn.sigmoid((hn * fc2r).sum(1, keepdims=True) + fc2b)


def _tc_compute(nbr_emb_all, nbr_feats2, nbr_times, time2, pn_all, pnode,
                ne_all, wt2, bt2, w1a, w1b, w1c, w1de, b12, w2a, w2b, b22,
                fc1a, fc1b, fc1b2, fc2r, fc2b2):
    row = lambda j, s: (s * JB + j, 0)
    const = lambda j, s: (0, 0)
    grid = (JB, 3)
    return pl.pallas_call(
        _tc_body,
        grid=grid,
        in_specs=[
            pl.BlockSpec((R, K * D), row),
            pl.BlockSpec((R, K * E), row),
            pl.BlockSpec((R, K), row),
            pl.BlockSpec((R, 1), lambda j, s: (j, 0)),
            pl.BlockSpec((L1, R, K * RP), lambda j, s: (0, s * JB + j, 0)),
            pl.BlockSpec((L1, R, RP), lambda j, s: (0, s * JB + j, 0)),
            pl.BlockSpec((L1, R, RP),
                         lambda j, s: (0, jnp.where(s == 0, JB, 0) + j, 0)),
            pl.BlockSpec((L1, R, RP), lambda j, s: (0, 2 * JB + j, 0)),
            pl.BlockSpec((R, D), row),
            pl.BlockSpec((1, T), const),
            pl.BlockSpec((1, T), const),
            pl.BlockSpec((D, HID), const),
            pl.BlockSpec((E, HID), const),
            pl.BlockSpec((T, HID), const),
            pl.BlockSpec((2 * L1, HID), const),
            pl.BlockSpec((1, HID), const),
            pl.BlockSpec((HID, OUT), const),
            pl.BlockSpec((D, OUT), const),
            pl.BlockSpec((1, OUT), const),
            pl.BlockSpec((OUT, OUT), const),
            pl.BlockSpec((OUT, OUT), const),
            pl.BlockSpec((1, OUT), const),
            pl.BlockSpec((1, OUT), const),
            pl.BlockSpec((1, 1), const),
        ],
        out_specs=[
            pl.BlockSpec((R, 1), lambda j, s: (j, 0)),
            pl.BlockSpec((R, 1), lambda j, s: (j, 0)),
        ],
        out_shape=[
            jax.ShapeDtypeStruct((B, 1), _f32),
            jax.ShapeDtypeStruct((B, 1), _f32),
        ],
        scratch_shapes=[
            pltpu.VMEM((R, OUT), _f32),
            pltpu.VMEM((R, OUT), _f32),
            pltpu.VMEM((R, OUT), _f32),
        ],
    )(nbr_emb_all, nbr_feats2, nbr_times, time2, pn_all, pnode, pnode, pnode,
      ne_all, wt2, bt2, w1a, w1b, w1c, w1de, b12, w2a, w2b, b22,
      fc1a, fc1b, fc1b2, fc2r, fc2b2)


def kernel(src, dst, neg, time, nbr_nids, nbr_times, nbr_feats,
           static_node_feat, wt, bt, W1, b1, W2, b2, fc1_w, fc1_b,
           fc2_w, fc2_b, P):
    idn = nbr_nids.reshape(-1).astype(jnp.int32)
    ido = jnp.concatenate([src, dst, neg]).astype(jnp.int32)
    lofs = jnp.arange(L1, dtype=jnp.int32)[:, None] * NN
    idpn = (lofs + idn[None, :]).reshape(-1)
    idpo = (lofs + ido[None, :]).reshape(-1)

    gf_nbr, gf_node, gp_nbr, gp_node = _make_sc_gather()(
        static_node_feat, P.reshape(L1 * NN, RP),
        idn.reshape(NW, CFN, CH), ido.reshape(NW, CFO, CH),
        idpn.reshape(NW, CPN, CH), idpo.reshape(NW, CPO, CH))

    nbr_emb_all = gf_nbr.reshape(3 * B, K * D)
    pn_all = gp_nbr.reshape(L1, 3 * B, K * RP)
    pnode = gp_node.reshape(L1, 3 * B, RP)

    pos, negv = _tc_compute(
        nbr_emb_all, nbr_feats.reshape(3 * B, K * E), nbr_times,
        time.reshape(B, 1), pn_all, pnode, gf_node,
        wt.reshape(1, T), bt.reshape(1, T),
        W1[:D], W1[D:D + E], W1[D + E:D + E + T], W1[D + E + T:],
        b1.reshape(1, HID), W2[:HID], W2[HID:], b2.reshape(1, OUT),
        fc1_w[:OUT], fc1_w[OUT:], fc1_b.reshape(1, OUT),
        fc2_w.reshape(1, OUT), fc2_b.reshape(1, 1))
    return pos.reshape(-1), negv.reshape(-1)


# speedup vs baseline: 4.7797x; 1.3462x over previous
import functools

import jax
import jax.numpy as jnp
from jax import lax
from jax.experimental import pallas as pl
from jax.experimental.pallas import tpu as pltpu
from jax.experimental.pallas import tpu_sc as plsc

NN = 100000
B = 4096
K = 20
D = 128
E = 16
T = 100
L1 = 3
RP = 64
HID = 128
OUT = 128
LRP = L1 * RP

NW = 32
CH = 64

NFN = 3 * B * K
NFO = 3 * B
NPN = L1 * NFN
NPO = L1 * NFO

CFN = NFN // (NW * CH)
CFO = NFO // (NW * CH)
CPN = NPN // (NW * CH)
CPO = NPO // (NW * CH)

R = 128
RK = R * K
JB = B // R

_f32 = jnp.float32

_COS_C = (0.9999999880426668, -0.4999998826125991, 0.041666477944581455,
          -0.0013887749113736198, 2.4768708072763377e-05,
          -2.7067459170587084e-07, 1.7202726782420442e-09)
_INV2PI = 0.15915494309189535
_TWOPI = 6.283185307179586


def _fast_cos(y):
    n = jnp.floor(y * _INV2PI + 0.5)
    r = y - n * _TWOPI
    u = r * r
    p = jnp.float32(_COS_C[6])
    for k in (5, 4, 3, 2, 1, 0):
        p = p * u + jnp.float32(_COS_C[k])
    return p


@functools.lru_cache(maxsize=1)
def _make_sc_gather():
    return functools.partial(
        pl.kernel,
        out_type=[
            jax.ShapeDtypeStruct((NFN, D), _f32),
            jax.ShapeDtypeStruct((NFO, D), _f32),
            jax.ShapeDtypeStruct((NPN, RP), _f32),
            jax.ShapeDtypeStruct((NPO, RP), _f32),
        ],
        mesh=plsc.VectorSubcoreMesh(core_axis_name="c", subcore_axis_name="s"),
        compiler_params=pltpu.CompilerParams(use_tc_tiling_on_sc=False),
        scratch_types=[
            pltpu.VMEM((CFN, CH), jnp.int32),
            pltpu.VMEM((CFO, CH), jnp.int32),
            pltpu.VMEM((CPN, CH), jnp.int32),
            pltpu.VMEM((CPO, CH), jnp.int32),
            pltpu.VMEM((CH, D), _f32),
            pltpu.VMEM((CH, RP), _f32),
            pltpu.SemaphoreType.DMA,
        ],
    )(_sc_gather_body)


def _sc_gather_body(feat_hbm, pflat_hbm, ifn_hbm, ifo_hbm, ipn_hbm, ipo_hbm,
                    ofn, ofo, opn, opo, ifn_v, ifo_v, ipn_v, ipo_v,
                    fbuf, pbuf, sem):
    wid = lax.axis_index("s") * 2 + lax.axis_index("c")
    pltpu.sync_copy(ifn_hbm.at[wid], ifn_v)
    pltpu.sync_copy(ifo_hbm.at[wid], ifo_v)
    pltpu.sync_copy(ipn_hbm.at[wid], ipn_v)
    pltpu.sync_copy(ipo_hbm.at[wid], ipo_v)

    def run(table, idx_v, nch, out, buf):
        base = wid * (nch * CH)

        def body(jc, carry):
            pltpu.async_copy(table.at[idx_v.at[jc]], buf, sem).wait()
            pltpu.sync_copy(buf, out.at[pl.ds(base + jc * CH, CH)])
            return carry

        lax.fori_loop(0, nch, body, 0)

    run(feat_hbm, ifn_v, CFN, ofn, fbuf)
    run(feat_hbm, ifo_v, CFO, ofo, fbuf)
    run(pflat_hbm, ipn_v, CPN, opn, pbuf)
    run(pflat_hbm, ipo_v, CPO, opo, pbuf)


def _tc_body(ne_ref, nf_ref, nt_ref, trep_ref,
             pn_ref, pu_ref, pv_ref, pv2_ref, nemb_ref,
             wt_ref, bt_ref, w1a_ref, w1b_ref, w1c_ref, w1dx_ref, w1ex_ref,
             b1_ref, w2a_ref, w2b_ref, b2_ref, fc1a_ref, fc1b_ref,
             fc1bias_ref, fc2r_ref, fc2b_ref,
             pos_ref, neg_ref,
             zsp_ref, zsn_ref, zdp_ref):
    s = pl.program_id(1)
    dn0 = (((0,), (0,)), ((), ()))

    dt = trep_ref[0] - nt_ref[0]
    te = _fast_cos(wt_ref[...] * dt + bt_ref[...])

    seg_m = lax.broadcasted_iota(jnp.int32, (RK, R), 0) // K
    seg_r = lax.broadcasted_iota(jnp.int32, (RK, R), 1)
    rep = jnp.where(seg_m == seg_r, 1.0, 0.0).astype(_f32)
    segt_r = lax.broadcasted_iota(jnp.int32, (R, RK), 0)
    segt_m = lax.broadcasted_iota(jnp.int32, (R, RK), 1) // K
    repk = jnp.where(segt_r == segt_m, jnp.float32(1.0 / K),
                     jnp.float32(0.0))

    pn = pn_ref[...]
    pu_rep = jnp.dot(rep, pu_ref[...], preferred_element_type=_f32)
    pv_rep = jnp.dot(rep, pv_ref[...], preferred_element_type=_f32)

    base = (jnp.dot(ne_ref[...], w1a_ref[...], preferred_element_type=_f32)
            + jnp.dot(nf_ref[...], w1b_ref[...], preferred_element_type=_f32)
            + lax.dot_general(te, w1c_ref[...], dn0,
                              preferred_element_type=_f32)
            + jnp.dot(pn * pu_rep, w1dx_ref[...],
                      preferred_element_type=_f32)
            + b1_ref[...])
    acc = base + jnp.dot(pn * pv_rep, w1ex_ref[...],
                         preferred_element_type=_f32)

    h = jnp.dot(repk, jax.nn.relu(acc), preferred_element_type=_f32)
    nemb = nemb_ref[...]
    w2a = w2a_ref[...]
    w2b = w2b_ref[...]
    b2 = b2_ref[...]
    z = (jnp.dot(h, w2a, preferred_element_type=_f32)
         + jnp.dot(nemb, w2b, preferred_element_type=_f32) + b2)

    @pl.when(s == 0)
    def _():
        zsp_ref[...] = z
        pv2_rep = jnp.dot(rep, pv2_ref[...], preferred_element_type=_f32)
        accn = base + jnp.dot(pn * pv2_rep, w1ex_ref[...],
                              preferred_element_type=_f32)
        hn = jnp.dot(repk, jax.nn.relu(accn), preferred_element_type=_f32)
        zsn_ref[...] = (jnp.dot(hn, w2a, preferred_element_type=_f32)
                        + jnp.dot(nemb, w2b, preferred_element_type=_f32)
                        + b2)

    @pl.when(s == 1)
    def _():
        zdp_ref[...] = z

    @pl.when(s == 2)
    def _():
        fc1a = fc1a_ref[...]
        fc1b = fc1b_ref[...]
        fc1bias = fc1bias_ref[...]
        fc2r = fc2r_ref[...]
        fc2b = fc2b_ref[...]
        hp = jax.nn.relu(jnp.dot(zsp_ref[...], fc1a,
                                 preferred_element_type=_f32)
                         + jnp.dot(zdp_ref[...], fc1b,
                                   preferred_element_type=_f32)
                         + fc1bias)
        pos_ref[...] = jax.nn.sigmoid((hp * fc2r).sum(1, keepdims=True)
                                      + fc2b)
        hn = jax.nn.relu(jnp.dot(zsn_ref[...], fc1a,
                                 preferred_element_type=_f32)
                         + jnp.dot(z, fc1b, preferred_element_type=_f32)
                         + fc1bias)
        neg_ref[...] = jax.nn.sigmoid((hn * fc2r).sum(1, keepdims=True)
                                      + fc2b)


def _tc_compute(ne2d, nf2d, nt3, trep3, pn2d, pnode2d, nemb2d,
                wt2, bt2, w1a, w1b, w1c, w1dx, w1ex, b12, w2a, w2b, b22,
                fc1a, fc1b, fc1b2, fc2r, fc2b2):
    row = lambda j, s: (s * JB + j, 0)
    const = lambda j, s: (0, 0)
    return pl.pallas_call(
        _tc_body,
        grid=(JB, 3),
        in_specs=[
            pl.BlockSpec((RK, D), row),
            pl.BlockSpec((RK, E), row),
            pl.BlockSpec((1, 1, RK), lambda j, s: (s * JB + j, 0, 0)),
            pl.BlockSpec((1, 1, RK), lambda j, s: (j, 0, 0)),
            pl.BlockSpec((RK, LRP), row),
            pl.BlockSpec((R, LRP), row),
            pl.BlockSpec((R, LRP),
                         lambda j, s: (jnp.where(s == 0, JB, 0) + j, 0)),
            pl.BlockSpec((R, LRP), lambda j, s: (2 * JB + j, 0)),
            pl.BlockSpec((R, D), row),
            pl.BlockSpec((T, 1), const),
            pl.BlockSpec((T, 1), const),
            pl.BlockSpec((D, HID), const),
            pl.BlockSpec((E, HID), const),
            pl.BlockSpec((T, HID), const),
            pl.BlockSpec((LRP, HID), const),
            pl.BlockSpec((LRP, HID), const),
            pl.BlockSpec((1, HID), const),
            pl.BlockSpec((HID, OUT), const),
            pl.BlockSpec((D, OUT), const),
            pl.BlockSpec((1, OUT), const),
            pl.BlockSpec((OUT, OUT), const),
            pl.BlockSpec((OUT, OUT), const),
            pl.BlockSpec((1, OUT), const),
            pl.BlockSpec((1, OUT), const),
            pl.BlockSpec((1, 1), const),
        ],
        out_specs=[
            pl.BlockSpec((R, 1), lambda j, s: (j, 0)),
            pl.BlockSpec((R, 1), lambda j, s: (j, 0)),
        ],
        out_shape=[
            jax.ShapeDtypeStruct((B, 1), _f32),
            jax.ShapeDtypeStruct((B, 1), _f32),
        ],
        scratch_shapes=[
            pltpu.VMEM((R, OUT), _f32),
            pltpu.VMEM((R, OUT), _f32),
            pltpu.VMEM((R, OUT), _f32),
        ],
    )(ne2d, nf2d, nt3, trep3, pn2d, pnode2d, pnode2d, pnode2d, nemb2d,
      wt2, bt2, w1a, w1b, w1c, w1dx, w1ex, b12, w2a, w2b, b22,
      fc1a, fc1b, fc1b2, fc2r, fc2b2)


def kernel(src, dst, neg, time, nbr_nids, nbr_times, nbr_feats,
           static_node_feat, wt, bt, W1, b1, W2, b2, fc1_w, fc1_b,
           fc2_w, fc2_b, P):
    idn = nbr_nids.reshape(-1).astype(jnp.int32)
    ido = jnp.concatenate([src, dst, neg]).astype(jnp.int32)
    lofs = jnp.arange(L1, dtype=jnp.int32)[None, :] * NN
    idpn = (idn[:, None] + lofs).reshape(-1)
    idpo = (ido[:, None] + lofs).reshape(-1)

    gf_nbr, gf_node, gp_nbr, gp_node = _make_sc_gather()(
        static_node_feat, P.reshape(L1 * NN, RP),
        idn.reshape(NW, CFN, CH), ido.reshape(NW, CFO, CH),
        idpn.reshape(NW, CPN, CH), idpo.reshape(NW, CPO, CH))

    pos, negv = _tc_compute(
        gf_nbr, nbr_feats.reshape(NFN, E),
        nbr_times.reshape(NFN // RK, 1, RK),
        jnp.repeat(time, K).reshape(B * K // RK, 1, RK),
        gp_nbr.reshape(NFN, LRP), gp_node.reshape(NFO, LRP), gf_node,
        wt.reshape(T, 1), bt.reshape(T, 1),
        W1[:D], W1[D:D + E], W1[D + E:D + E + T],
        jnp.repeat(W1[D + E + T:D + E + T + L1], RP, axis=0),
        jnp.repeat(W1[D + E + T + L1:], RP, axis=0),
        b1.reshape(1, HID), W2[:HID], W2[HID:], b2.reshape(1, OUT),
        fc1_w[:OUT], fc1_w[OUT:], fc1_b.reshape(1, OUT),
        fc2_w.reshape(1, OUT), fc2_b.reshape(1, 1))
    return pos.reshape(-1), negv.reshape(-1)


# speedup vs baseline: 6.3088x; 1.3199x over previous
import functools

import jax
import jax.numpy as jnp
from jax import lax
from jax.experimental import pallas as pl
from jax.experimental.pallas import tpu as pltpu
from jax.experimental.pallas import tpu_sc as plsc

NN = 100000
B = 4096
K = 20
D = 128
E = 16
T = 100
L1 = 3
RP = 64
HID = 128
OUT = 128
LRP = L1 * RP

NW = 32
CH = 64

NFN = 3 * B * K
NFO = 3 * B
NPN = L1 * NFN
NPO = L1 * NFO

FPW = NFN // NW
OPW = NFO // NW
CFN = FPW // CH
CFO = OPW // CH

R = 128
RK = R * K
JB = B // R
NBN = NFN // RK
NBO = NFO // R

_f32 = jnp.float32

_COS_C = (0.9999999880426668, -0.4999998826125991, 0.041666477944581455,
          -0.0013887749113736198, 2.4768708072763377e-05,
          -2.7067459170587084e-07, 1.7202726782420442e-09)
_INV2PI = 0.15915494309189535
_TWOPI = 6.283185307179586


def _fast_cos(y):
    n = jnp.floor(y * _INV2PI + 0.5)
    r = y - n * _TWOPI
    u = r * r
    p = jnp.float32(_COS_C[6])
    for k in (5, 4, 3, 2, 1, 0):
        p = p * u + jnp.float32(_COS_C[k])
    return p


@functools.lru_cache(maxsize=1)
def _make_sc_gather():
    return functools.partial(
        pl.kernel,
        out_type=[
            jax.ShapeDtypeStruct((NFN, D), _f32),
            jax.ShapeDtypeStruct((NFO, D), _f32),
            jax.ShapeDtypeStruct((NPN, RP), _f32),
            jax.ShapeDtypeStruct((NPO, RP), _f32),
        ],
        mesh=plsc.VectorSubcoreMesh(core_axis_name="c", subcore_axis_name="s"),
        compiler_params=pltpu.CompilerParams(use_tc_tiling_on_sc=False),
        scratch_types=[
            pltpu.VMEM((FPW,), jnp.int32),
            pltpu.VMEM((OPW,), jnp.int32),
            pltpu.VMEM((CH,), jnp.int32),
            pltpu.VMEM((CH,), jnp.int32),
            pltpu.VMEM((CH, D), _f32),
            pltpu.VMEM((CH, D), _f32),
            pltpu.VMEM((CH, RP), _f32),
            pltpu.VMEM((CH, RP), _f32),
            pltpu.SemaphoreType.DMA,
            pltpu.SemaphoreType.DMA,
        ],
    )(_sc_gather_body)


def _sc_gather_body(feat_hbm, pflat_hbm, idn_hbm, ido_hbm,
                    ofn, ofo, opn, opo,
                    ifn_v, ifo_v, pidx0, pidx1, fbuf0, fbuf1,
                    pbuf0, pbuf1, sem0, sem1):
    wid = lax.axis_index("s") * 2 + lax.axis_index("c")
    pltpu.sync_copy(idn_hbm.at[pl.ds(wid * FPW, FPW)], ifn_v)
    pltpu.sync_copy(ido_hbm.at[pl.ds(wid * OPW, OPW)], ifo_v)

    def gather_direct(table, idx_v, nch, out, base, buf0, buf1):
        def start(jc, buf, sem):
            return pltpu.async_copy(
                table.at[idx_v.at[pl.ds(jc * CH, CH)]], buf, sem)

        start(0, buf0, sem0)

        def body(g, carry):
            c0 = 2 * g
            cp1 = start(c0 + 1, buf1, sem1)
            pltpu.make_async_copy(
                table.at[idx_v.at[pl.ds(c0 * CH, CH)]], buf0, sem0).wait()
            pltpu.sync_copy(buf0, out.at[pl.ds(base + c0 * CH, CH)])

            @pl.when(c0 + 2 < nch)
            def _():
                start(c0 + 2, buf0, sem0)

            cp1.wait()
            pltpu.sync_copy(buf1, out.at[pl.ds(base + (c0 + 1) * CH, CH)])
            return carry

        lax.fori_loop(0, nch // 2, body, 0)

    def gather_sketch(idx_v, nch, rows_total, out, lofs, buf0, buf1):
        def stage(jc, pidx):
            for k4 in range(CH // 16):
                pidx[pl.ds(k4 * 16, 16)] = (
                    idx_v[pl.ds(jc * CH + k4 * 16, 16)] + lofs)

        def start(pidx, buf, sem):
            return pltpu.async_copy(pflat_hbm.at[pidx], buf, sem)

        base = wid * rows_total

        stage(0, pidx0)
        start(pidx0, buf0, sem0)

        def body(g, carry):
            c0 = 2 * g
            stage(c0 + 1, pidx1)
            cp1 = start(pidx1, buf1, sem1)
            pltpu.make_async_copy(pflat_hbm.at[pidx0], buf0, sem0).wait()
            pltpu.sync_copy(buf0, out.at[pl.ds(base + c0 * CH, CH)])

            @pl.when(c0 + 2 < nch)
            def _():
                stage(c0 + 2, pidx0)
                start(pidx0, buf0, sem0)

            cp1.wait()
            pltpu.sync_copy(buf1, out.at[pl.ds(base + (c0 + 1) * CH, CH)])
            return carry

        lax.fori_loop(0, nch // 2, body, 0)

    gather_direct(feat_hbm, ifn_v, CFN, ofn, wid * FPW, fbuf0, fbuf1)
    gather_direct(feat_hbm, ifo_v, CFO, ofo, wid * OPW, fbuf0, fbuf1)
    for l in range(L1):
        lofs = jnp.int32(l * NN)
        gather_sketch(ifn_v, CFN, FPW, opn.at[pl.ds(l * NFN, NFN)], lofs,
                      pbuf0, pbuf1)
    for l in range(L1):
        lofs = jnp.int32(l * NN)
        gather_sketch(ifo_v, CFO, OPW, opo.at[pl.ds(l * NFO, NFO)], lofs,
                      pbuf0, pbuf1)


def _tc_body(ne_ref, nf_ref, nt_ref, trep_ref,
             pn0_ref, pn1_ref, pn2_ref,
             pu0_ref, pu1_ref, pu2_ref,
             pv0_ref, pv1_ref, pv2_ref,
             pw0_ref, pw1_ref, pw2_ref,
             nemb_ref,
             wt_ref, bt_ref, w1a_ref, w1b_ref, w1c_ref, w1dx_ref, w1ex_ref,
             b1_ref, w2a_ref, w2b_ref, b2_ref, fc1a_ref, fc1b_ref,
             fc1bias_ref, fc2r_ref, fc2b_ref,
             pos_ref, neg_ref,
             zsp_ref, zsn_ref, zdp_ref):
    s = pl.program_id(1)
    dn0 = (((0,), (0,)), ((), ()))
    pns = (pn0_ref, pn1_ref, pn2_ref)
    pus = (pu0_ref, pu1_ref, pu2_ref)
    pvs = (pv0_ref, pv1_ref, pv2_ref)
    pws = (pw0_ref, pw1_ref, pw2_ref)

    dt = trep_ref[0] - nt_ref[0]
    te = _fast_cos(wt_ref[...] * dt + bt_ref[...])

    def rep_rows(x):
        return jnp.broadcast_to(x[:, None, :], (R, K, RP)).reshape(RK, RP)

    base = (jnp.dot(ne_ref[...], w1a_ref[...], preferred_element_type=_f32)
            + jnp.dot(nf_ref[...], w1b_ref[...], preferred_element_type=_f32)
            + lax.dot_general(te, w1c_ref[...], dn0,
                              preferred_element_type=_f32)
            + b1_ref[...])
    accv = jnp.zeros((RK, HID), _f32)
    for l in range(L1):
        pn = pns[l][...]
        base = base + jnp.dot(pn * rep_rows(pus[l][...]),
                              w1dx_ref[pl.ds(l * RP, RP)],
                              preferred_element_type=_f32)
        accv = accv + jnp.dot(pn * rep_rows(pvs[l][...]),
                              w1ex_ref[pl.ds(l * RP, RP)],
                              preferred_element_type=_f32)
    acc = base + accv

    def pool(a):
        return jnp.mean(jax.nn.relu(a).reshape(R, K, HID), axis=1)

    h = pool(acc)
    nemb = nemb_ref[...]
    w2a = w2a_ref[...]
    w2b = w2b_ref[...]
    b2 = b2_ref[...]
    zn_part = jnp.dot(nemb, w2b, preferred_element_type=_f32) + b2
    z = jnp.dot(h, w2a, preferred_element_type=_f32) + zn_part

    @pl.when(s == 0)
    def _():
        zsp_ref[...] = z
        accn = base
        for l in range(L1):
            accn = accn + jnp.dot(pns[l][...] * rep_rows(pws[l][...]),
                                  w1ex_ref[pl.ds(l * RP, RP)],
                                  preferred_element_type=_f32)
        hn = pool(accn)
        zsn_ref[...] = jnp.dot(hn, w2a, preferred_element_type=_f32) + zn_part

    @pl.when(s == 1)
    def _():
        zdp_ref[...] = z

    @pl.when(s == 2)
    def _():
        fc1a = fc1a_ref[...]
        fc1b = fc1b_ref[...]
        fc1bias = fc1bias_ref[...]
        fc2r = fc2r_ref[...]
        fc2b = fc2b_ref[...]
        hp = jax.nn.relu(jnp.dot(zsp_ref[...], fc1a,
                                 preferred_element_type=_f32)
                         + jnp.dot(zdp_ref[...], fc1b,
                                   preferred_element_type=_f32)
                         + fc1bias)
        pos_ref[...] = jax.nn.sigmoid((hp * fc2r).sum(1, keepdims=True)
                                      + fc2b)
        hn = jax.nn.relu(jnp.dot(zsn_ref[...], fc1a,
                                 preferred_element_type=_f32)
                         + jnp.dot(z, fc1b, preferred_element_type=_f32)
                         + fc1bias)
        neg_ref[...] = jax.nn.sigmoid((hn * fc2r).sum(1, keepdims=True)
                                      + fc2b)


def _tc_compute(ne2d, nf2d, nt3, trep3, pn2d, pnode2d, nemb2d,
                wt2, bt2, w1a, w1b, w1c, w1dx, w1ex, b12, w2a, w2b, b22,
                fc1a, fc1b, fc1b2, fc2r, fc2b2):
    row = lambda j, s: (s * JB + j, 0)
    const = lambda j, s: (0, 0)

    def pn_spec(l):
        return pl.BlockSpec((RK, RP), lambda j, s, l=l: (l * NBN + s * JB + j, 0))

    def pu_spec(l):
        return pl.BlockSpec((R, RP), lambda j, s, l=l: (l * NBO + s * JB + j, 0))

    def pv_spec(l):
        return pl.BlockSpec(
            (R, RP),
            lambda j, s, l=l: (l * NBO + jnp.where(s == 0, JB, 0) + j, 0))

    def pw_spec(l):
        return pl.BlockSpec((R, RP),
                            lambda j, s, l=l: (l * NBO + 2 * JB + j, 0))

    return pl.pallas_call(
        _tc_body,
        grid=(JB, 3),
        in_specs=[
            pl.BlockSpec((RK, D), row),
            pl.BlockSpec((RK, E), row),
            pl.BlockSpec((1, 1, RK), lambda j, s: (s * JB + j, 0, 0)),
            pl.BlockSpec((1, 1, RK), lambda j, s: (j, 0, 0)),
            pn_spec(0), pn_spec(1), pn_spec(2),
            pu_spec(0), pu_spec(1), pu_spec(2),
            pv_spec(0), pv_spec(1), pv_spec(2),
            pw_spec(0), pw_spec(1), pw_spec(2),
            pl.BlockSpec((R, D), row),
            pl.BlockSpec((T, 1), const),
            pl.BlockSpec((T, 1), const),
            pl.BlockSpec((D, HID), const),
            pl.BlockSpec((E, HID), const),
            pl.BlockSpec((T, HID), const),
            pl.BlockSpec((LRP, HID), const),
            pl.BlockSpec((LRP, HID), const),
            pl.BlockSpec((1, HID), const),
            pl.BlockSpec((HID, OUT), const),
            pl.BlockSpec((D, OUT), const),
            pl.BlockSpec((1, OUT), const),
            pl.BlockSpec((OUT, OUT), const),
            pl.BlockSpec((OUT, OUT), const),
            pl.BlockSpec((1, OUT), const),
            pl.BlockSpec((1, OUT), const),
            pl.BlockSpec((1, 1), const),
        ],
        out_specs=[
            pl.BlockSpec((R, 1), lambda j, s: (j, 0)),
            pl.BlockSpec((R, 1), lambda j, s: (j, 0)),
        ],
        out_shape=[
            jax.ShapeDtypeStruct((B, 1), _f32),
            jax.ShapeDtypeStruct((B, 1), _f32),
        ],
        scratch_shapes=[
            pltpu.VMEM((R, OUT), _f32),
            pltpu.VMEM((R, OUT), _f32),
            pltpu.VMEM((R, OUT), _f32),
        ],
    )(ne2d, nf2d, nt3, trep3,
      pn2d, pn2d, pn2d,
      pnode2d, pnode2d, pnode2d,
      pnode2d, pnode2d, pnode2d,
      pnode2d, pnode2d, pnode2d,
      nemb2d,
      wt2, bt2, w1a, w1b, w1c, w1dx, w1ex, b12, w2a, w2b, b22,
      fc1a, fc1b, fc1b2, fc2r, fc2b2)


def kernel(src, dst, neg, time, nbr_nids, nbr_times, nbr_feats,
           static_node_feat, wt, bt, W1, b1, W2, b2, fc1_w, fc1_b,
           fc2_w, fc2_b, P):
    idn = nbr_nids.reshape(-1).astype(jnp.int32)
    ido = jnp.concatenate([src, dst, neg]).astype(jnp.int32)

    gf_nbr, gf_node, gp_nbr, gp_node = _make_sc_gather()(
        static_node_feat, P.reshape(L1 * NN, RP), idn, ido)

    pos, negv = _tc_compute(
        gf_nbr, nbr_feats.reshape(NFN, E),
        nbr_times.reshape(NFN // RK, 1, RK),
        jnp.repeat(time, K).reshape(B * K // RK, 1, RK),
        gp_nbr, gp_node, gf_node,
        wt.reshape(T, 1), bt.reshape(T, 1),
        W1[:D], W1[D:D + E], W1[D + E:D + E + T],
        jnp.repeat(W1[D + E + T:D + E + T + L1], RP, axis=0),
        jnp.repeat(W1[D + E + T + L1:], RP, axis=0),
        b1.reshape(1, HID), W2[:HID], W2[HID:], b2.reshape(1, OUT),
        fc1_w[:OUT], fc1_w[OUT:], fc1_b.reshape(1, OUT),
        fc2_w.reshape(1, OUT), fc2_b.reshape(1, 1))
    return pos.reshape(-1), negv.reshape(-1)


# speedup vs baseline: 8.2618x; 1.3096x over previous
import functools

import jax
import jax.numpy as jnp
from jax import lax
from jax.experimental import pallas as pl
from jax.experimental.pallas import tpu as pltpu
from jax.experimental.pallas import tpu_sc as plsc

NN = 100000
B = 4096
K = 20
D = 128
E = 16
T = 100
L1 = 3
RP = 64
HID = 128
OUT = 128
LRP = L1 * RP

NW = 32
CH = 64

NFN = 3 * B * K
NFO = 3 * B
NPN = L1 * NFN
NPO = L1 * NFO

FPW = NFN // NW
OPW = NFO // NW
CFN = FPW // CH
CFO = OPW // CH

R = 128
RK = R * K
JB = B // R
NBN = NFN // RK
NBO = NFO // R

_f32 = jnp.float32

_COS_C = (0.9999999880426668, -0.4999998826125991, 0.041666477944581455,
          -0.0013887749113736198, 2.4768708072763377e-05,
          -2.7067459170587084e-07, 1.7202726782420442e-09)
_INV2PI = 0.15915494309189535
_TWOPI = 6.283185307179586


def _fast_cos(y):
    n = jnp.floor(y * _INV2PI + 0.5)
    r = y - n * _TWOPI
    u = r * r
    p = jnp.float32(_COS_C[6])
    for k in (5, 4, 3, 2, 1, 0):
        p = p * u + jnp.float32(_COS_C[k])
    return p


@functools.lru_cache(maxsize=1)
def _make_sc_gather():
    return functools.partial(
        pl.kernel,
        out_type=[
            jax.ShapeDtypeStruct((NFN, D), _f32),
            jax.ShapeDtypeStruct((NFO, D), _f32),
            jax.ShapeDtypeStruct((NPN // 2, 2 * RP), _f32),
            jax.ShapeDtypeStruct((NPO // 2, 2 * RP), _f32),
        ],
        mesh=plsc.VectorSubcoreMesh(core_axis_name="c", subcore_axis_name="s"),
        compiler_params=pltpu.CompilerParams(use_tc_tiling_on_sc=False),
        scratch_types=[
            pltpu.VMEM((FPW,), jnp.int32),
            pltpu.VMEM((OPW,), jnp.int32),
            pltpu.VMEM((CH,), jnp.int32),
            pltpu.VMEM((CH,), jnp.int32),
            pltpu.VMEM((CH, D), _f32),
            pltpu.VMEM((CH, D), _f32),
            pltpu.VMEM((CH, RP), _f32),
            pltpu.VMEM((CH, RP), _f32),
            pltpu.SemaphoreType.DMA,
            pltpu.SemaphoreType.DMA,
        ],
    )(_sc_gather_body)


def _sc_gather_body(feat_hbm, pflat_hbm, idn_hbm, ido_hbm,
                    ofn, ofo, opn, opo,
                    ifn_v, ifo_v, pidx0, pidx1, fbuf0, fbuf1,
                    pbuf0, pbuf1, sem0, sem1):
    wid = lax.axis_index("s") * 2 + lax.axis_index("c")
    pltpu.sync_copy(idn_hbm.at[pl.ds(wid * FPW, FPW)], ifn_v)
    pltpu.sync_copy(ido_hbm.at[pl.ds(wid * OPW, OPW)], ifo_v)

    def gather_direct(table, idx_v, nch, out, base, buf0, buf1):
        def start(jc, buf, sem):
            return pltpu.async_copy(
                table.at[idx_v.at[pl.ds(jc * CH, CH)]], buf, sem)

        start(0, buf0, sem0)

        def body(g, carry):
            c0 = 2 * g
            cp1 = start(c0 + 1, buf1, sem1)
            pltpu.make_async_copy(
                table.at[idx_v.at[pl.ds(c0 * CH, CH)]], buf0, sem0).wait()
            pltpu.sync_copy(buf0, out.at[pl.ds(base + c0 * CH, CH)])

            @pl.when(c0 + 2 < nch)
            def _():
                start(c0 + 2, buf0, sem0)

            cp1.wait()
            pltpu.sync_copy(buf1, out.at[pl.ds(base + (c0 + 1) * CH, CH)])
            return carry

        lax.fori_loop(0, nch // 2, body, 0)

    def gather_sketch(idx_v, nch, cpb, pair, out, lofs, buf0, buf1):
        def stage(jc, pidx):
            for k4 in range(CH // 16):
                pidx[pl.ds(k4 * 16, 16)] = (
                    idx_v[pl.ds(jc * CH + k4 * 16, 16)] + lofs)

        def start(pidx, buf, sem):
            return pltpu.async_copy(pflat_hbm.at[pidx], buf, sem)

        base = wid * (nch // 2) * CH

        def writeback(jc, buf):
            t = jc // cpb
            lc = jc - t * cpb
            half = lc // (cpb // 2)
            within = lc - half * (cpb // 2)
            rowbase = base + t * pair + within * CH
            pltpu.sync_copy(
                buf, out.at[pl.ds(rowbase, CH), pl.ds(half * RP, RP)])

        stage(0, pidx0)
        start(pidx0, buf0, sem0)

        def body(g, carry):
            c0 = 2 * g
            stage(c0 + 1, pidx1)
            cp1 = start(pidx1, buf1, sem1)
            pltpu.make_async_copy(pflat_hbm.at[pidx0], buf0, sem0).wait()
            writeback(c0, buf0)

            @pl.when(c0 + 2 < nch)
            def _():
                stage(c0 + 2, pidx0)
                start(pidx0, buf0, sem0)

            cp1.wait()
            writeback(c0 + 1, buf1)
            return carry

        lax.fori_loop(0, nch // 2, body, 0)

    gather_direct(feat_hbm, ifn_v, CFN, ofn, wid * FPW, fbuf0, fbuf1)
    gather_direct(feat_hbm, ifo_v, CFO, ofo, wid * OPW, fbuf0, fbuf1)
    for l in range(L1):
        lofs = jnp.int32(l * NN)
        gather_sketch(ifn_v, CFN, RK // CH, RK // 2,
                      opn.at[pl.ds(l * (NFN // 2), NFN // 2)], lofs,
                      pbuf0, pbuf1)
    for l in range(L1):
        lofs = jnp.int32(l * NN)
        gather_sketch(ifo_v, CFO, R // CH, R // 2,
                      opo.at[pl.ds(l * (NFO // 2), NFO // 2)], lofs,
                      pbuf0, pbuf1)


def _tc_body(ne_ref, nf_ref, nt_ref, trep_ref,
             pn0_ref, pn1_ref, pn2_ref,
             pu0_ref, pu1_ref, pu2_ref,
             pv0_ref, pv1_ref, pv2_ref,
             pw0_ref, pw1_ref, pw2_ref,
             nemb_ref,
             wt_ref, bt_ref, w1a_ref, w1b_ref, w1c_ref, w1dx_ref, w1ex_ref,
             b1_ref, w2a_ref, w2b_ref, b2_ref, fc1a_ref, fc1b_ref,
             fc1bias_ref, fc2r_ref, fc2b_ref,
             pos_ref, neg_ref,
             zsp_ref, zsn_ref, zdp_ref):
    s = pl.program_id(1)
    dn0 = (((0,), (0,)), ((), ()))
    pns = (pn0_ref, pn1_ref, pn2_ref)
    pus = (pu0_ref, pu1_ref, pu2_ref)
    pvs = (pv0_ref, pv1_ref, pv2_ref)
    pws = (pw0_ref, pw1_ref, pw2_ref)

    dt = trep_ref[0] - nt_ref[0]
    te = _fast_cos(wt_ref[...] * dt + bt_ref[...])

    def rep_rows(x):
        return jnp.broadcast_to(x[:, None, :], (R, K, RP)).reshape(RK, RP)

    def unpair(x, n):
        return jnp.concatenate([x[:, :RP], x[:, RP:]], axis=0)

    base = (jnp.dot(ne_ref[...], w1a_ref[...], preferred_element_type=_f32)
            + jnp.dot(nf_ref[...], w1b_ref[...], preferred_element_type=_f32)
            + lax.dot_general(te, w1c_ref[...], dn0,
                              preferred_element_type=_f32)
            + b1_ref[...])
    accv = jnp.zeros((RK, HID), _f32)
    for l in range(L1):
        pn = unpair(pns[l][...], RK)
        base = base + jnp.dot(pn * rep_rows(unpair(pus[l][...], R)),
                              w1dx_ref[pl.ds(l * RP, RP)],
                              preferred_element_type=_f32)
        accv = accv + jnp.dot(pn * rep_rows(unpair(pvs[l][...], R)),
                              w1ex_ref[pl.ds(l * RP, RP)],
                              preferred_element_type=_f32)
    acc = base + accv

    def pool(a):
        return jnp.mean(jax.nn.relu(a).reshape(R, K, HID), axis=1)

    h = pool(acc)
    nemb = nemb_ref[...]
    w2a = w2a_ref[...]
    w2b = w2b_ref[...]
    b2 = b2_ref[...]
    zn_part = jnp.dot(nemb, w2b, preferred_element_type=_f32) + b2
    z = jnp.dot(h, w2a, preferred_element_type=_f32) + zn_part

    @pl.when(s == 0)
    def _():
        zsp_ref[...] = z
        accn = base
        for l in range(L1):
            accn = accn + jnp.dot(
                unpair(pns[l][...], RK)
                * rep_rows(unpair(pws[l][...], R)),
                w1ex_ref[pl.ds(l * RP, RP)],
                preferred_element_type=_f32)
        hn = pool(accn)
        zsn_ref[...] = jnp.dot(hn, w2a, preferred_element_type=_f32) + zn_part

    @pl.when(s == 1)
    def _():
        zdp_ref[...] = z

    @pl.when(s == 2)
    def _():
        fc1a = fc1a_ref[...]
        fc1b = fc1b_ref[...]
        fc1bias = fc1bias_ref[...]
        fc2r = fc2r_ref[...]
        fc2b = fc2b_ref[...]
        hp = jax.nn.relu(jnp.dot(zsp_ref[...], fc1a,
                                 preferred_element_type=_f32)
                         + jnp.dot(zdp_ref[...], fc1b,
                                   preferred_element_type=_f32)
                         + fc1bias)
        pos_ref[...] = jax.nn.sigmoid((hp * fc2r).sum(1, keepdims=True)
                                      + fc2b)
        hn = jax.nn.relu(jnp.dot(zsn_ref[...], fc1a,
                                 preferred_element_type=_f32)
                         + jnp.dot(z, fc1b, preferred_element_type=_f32)
                         + fc1bias)
        neg_ref[...] = jax.nn.sigmoid((hn * fc2r).sum(1, keepdims=True)
                                      + fc2b)


def _tc_compute(ne2d, nf2d, nt3, trep3, pn2d, pnode2d, nemb2d,
                wt2, bt2, w1a, w1b, w1c, w1dx, w1ex, b12, w2a, w2b, b22,
                fc1a, fc1b, fc1b2, fc2r, fc2b2):
    row = lambda j, s: (s * JB + j, 0)
    const = lambda j, s: (0, 0)

    def pn_spec(l):
        return pl.BlockSpec((RK // 2, 2 * RP),
                            lambda j, s, l=l: (l * NBN + s * JB + j, 0))

    def pu_spec(l):
        return pl.BlockSpec((R // 2, 2 * RP),
                            lambda j, s, l=l: (l * NBO + s * JB + j, 0))

    def pv_spec(l):
        return pl.BlockSpec(
            (R // 2, 2 * RP),
            lambda j, s, l=l: (l * NBO + jnp.where(s == 0, JB, 0) + j, 0))

    def pw_spec(l):
        return pl.BlockSpec((R // 2, 2 * RP),
                            lambda j, s, l=l: (l * NBO + 2 * JB + j, 0))

    return pl.pallas_call(
        _tc_body,
        grid=(JB, 3),
        in_specs=[
            pl.BlockSpec((RK, D), row),
            pl.BlockSpec((RK, E), row),
            pl.BlockSpec((1, 1, RK), lambda j, s: (s * JB + j, 0, 0)),
            pl.BlockSpec((1, 1, RK), lambda j, s: (j, 0, 0)),
            pn_spec(0), pn_spec(1), pn_spec(2),
            pu_spec(0), pu_spec(1), pu_spec(2),
            pv_spec(0), pv_spec(1), pv_spec(2),
            pw_spec(0), pw_spec(1), pw_spec(2),
            pl.BlockSpec((R, D), row),
            pl.BlockSpec((T, 1), const),
            pl.BlockSpec((T, 1), const),
            pl.BlockSpec((D, HID), const),
            pl.BlockSpec((E, HID), const),
            pl.BlockSpec((T, HID), const),
            pl.BlockSpec((LRP, HID), const),
            pl.BlockSpec((LRP, HID), const),
            pl.BlockSpec((1, HID), const),
            pl.BlockSpec((HID, OUT), const),
            pl.BlockSpec((D, OUT), const),
            pl.BlockSpec((1, OUT), const),
            pl.BlockSpec((OUT, OUT), const),
            pl.BlockSpec((OUT, OUT), const),
            pl.BlockSpec((1, OUT), const),
            pl.BlockSpec((1, OUT), const),
            pl.BlockSpec((1, 1), const),
        ],
        out_specs=[
            pl.BlockSpec((R, 1), lambda j, s: (j, 0)),
            pl.BlockSpec((R, 1), lambda j, s: (j, 0)),
        ],
        out_shape=[
            jax.ShapeDtypeStruct((B, 1), _f32),
            jax.ShapeDtypeStruct((B, 1), _f32),
        ],
        scratch_shapes=[
            pltpu.VMEM((R, OUT), _f32),
            pltpu.VMEM((R, OUT), _f32),
            pltpu.VMEM((R, OUT), _f32),
        ],
    )(ne2d, nf2d, nt3, trep3,
      pn2d, pn2d, pn2d,
      pnode2d, pnode2d, pnode2d,
      pnode2d, pnode2d, pnode2d,
      pnode2d, pnode2d, pnode2d,
      nemb2d,
      wt2, bt2, w1a, w1b, w1c, w1dx, w1ex, b12, w2a, w2b, b22,
      fc1a, fc1b, fc1b2, fc2r, fc2b2)


def kernel(src, dst, neg, time, nbr_nids, nbr_times, nbr_feats,
           static_node_feat, wt, bt, W1, b1, W2, b2, fc1_w, fc1_b,
           fc2_w, fc2_b, P):
    idn = nbr_nids.reshape(-1).astype(jnp.int32)
    ido = jnp.concatenate([src, dst, neg]).astype(jnp.int32)

    gf_nbr, gf_node, gp_nbr, gp_node = _make_sc_gather()(
        static_node_feat, P.reshape(L1 * NN, RP), idn, ido)

    pos, negv = _tc_compute(
        gf_nbr, nbr_feats.reshape(NFN, E),
        nbr_times.reshape(NFN // RK, 1, RK),
        jnp.repeat(time, K).reshape(B * K // RK, 1, RK),
        gp_nbr, gp_node, gf_node,
        wt.reshape(T, 1), bt.reshape(T, 1),
        W1[:D], W1[D:D + E], W1[D + E:D + E + T],
        jnp.repeat(W1[D + E + T:D + E + T + L1], RP, axis=0),
        jnp.repeat(W1[D + E + T + L1:], RP, axis=0),
        b1.reshape(1, HID), W2[:HID], W2[HID:], b2.reshape(1, OUT),
        fc1_w[:OUT], fc1_w[OUT:], fc1_b.reshape(1, OUT),
        fc2_w.reshape(1, OUT), fc2_b.reshape(1, 1))
    return pos.reshape(-1), negv.reshape(-1)
